# baseline (speedup 1.0000x reference)
import math

import jax
import jax.numpy as jnp
from jax.experimental import pallas as pl
from jax.experimental.pallas import tpu as pltpu

B, S, D_IN, D_MODEL, H, L, KNN = 16, 128, 64, 256, 8, 2, 16
D_FF = 1024
DH = D_MODEL // H
N = B * S
ROW_BLK = 128
NUM_ROW_BLKS = N // ROW_BLK
F32 = jnp.float32
BF16 = jnp.bfloat16


def _bdot(a, b_ref_val):
    return jnp.dot(a.astype(BF16), b_ref_val,
                   preferred_element_type=F32)


def _layernorm(x, g, b):
    m = jnp.mean(x, axis=-1, keepdims=True)
    v = jnp.mean((x - m) ** 2, axis=-1, keepdims=True)
    return (x - m) * jax.lax.rsqrt(v + 1e-5) * g + b


NB = 4


N_FWD_STEPS = B // NB


def _fused_kernel(x_ref, win_ref, bin_ref, wqkv_ref, bqkv_ref, wo_ref, bo_ref,
                  w1_ref, b1_ref, w2_ref, b2_ref, g1_ref, be1_ref, g2_ref,
                  be2_ref, wdown_ref, bdown_ref, out_ref, loss_ref,
                  xrec_scratch):
    i = pl.program_id(0)

    @pl.when(i < N_FWD_STEPS)
    def _():
        _fwd_step(i, x_ref, win_ref, bin_ref, wqkv_ref, bqkv_ref, wo_ref,
                  bo_ref, w1_ref, b1_ref, w2_ref, b2_ref, g1_ref, be1_ref,
                  g2_ref, be2_ref, wdown_ref, bdown_ref, out_ref,
                  xrec_scratch)

    @pl.when(i >= N_FWD_STEPS)
    def _():
        _knn_step(i - N_FWD_STEPS, xrec_scratch, loss_ref)


def _fwd_step(i, x_ref, win_ref, bin_ref, wqkv_ref, bqkv_ref, wo_ref, bo_ref,
              w1_ref, b1_ref, w2_ref, b2_ref, g1_ref, be1_ref, g2_ref,
              be2_ref, wdown_ref, bdown_ref, out_ref, xrec_scratch):
    M = NB * S
    xb = x_ref[...].reshape(M, D_IN)
    h = _bdot(xb, win_ref[...]) + bin_ref[...]
    sel = (jax.lax.broadcasted_iota(jnp.int32, (H * S, H), 0) // S
           == jax.lax.broadcasted_iota(jnp.int32, (H * S, H), 1)).astype(BF16)
    rep = (jax.lax.broadcasted_iota(jnp.int32, (H, D_MODEL), 0)
           == jax.lax.broadcasted_iota(jnp.int32, (H, D_MODEL), 1) // DH
           ).astype(BF16)
    for l in range(L):
        qkv = _bdot(h, wqkv_ref[l]) + bqkv_ref[l]
        qkv_b = qkv.astype(BF16)
        o_rows = []
        for b in range(NB):
            rs = slice(b * S, (b + 1) * S)
            es, avs = [], []
            for hh in range(H):
                qs = slice(hh * DH, (hh + 1) * DH)
                ks = slice(D_MODEL + hh * DH, D_MODEL + (hh + 1) * DH)
                vs = slice(2 * D_MODEL + hh * DH, 2 * D_MODEL + (hh + 1) * DH)
                qh, kh, vh = qkv_b[rs, qs], qkv_b[rs, ks], qkv_b[rs, vs]
                s = jax.lax.dot_general(qh, kh, (((1,), (1,)), ((), ())),
                                        preferred_element_type=F32)
                e = jnp.exp(s).astype(BF16)
                es.append(e)
                avs.append(jnp.dot(e, vh, preferred_element_type=F32))
            e_cat = jnp.concatenate(es, axis=1)
            sums = jnp.dot(e_cat, sel, preferred_element_type=F32)
            r_rep = jnp.dot((1.0 / sums).astype(BF16), rep,
                            preferred_element_type=F32)
            o_rows.append(jnp.concatenate(avs, axis=1) * r_rep)
        o = jnp.concatenate(o_rows, axis=0)
        h = _layernorm(h + _bdot(o, wo_ref[l]) + bo_ref[l],
                       g1_ref[l], be1_ref[l])
        ff = jnp.maximum(_bdot(h, w1_ref[l]) + b1_ref[l], 0.0)
        ff = _bdot(ff, w2_ref[l]) + b2_ref[l]
        h = _layernorm(h + ff, g2_ref[l], be2_ref[l])
    xr = _bdot(h, wdown_ref[...]) + bdown_ref[...]
    out_ref[...] = xr.reshape(NB, S, D_IN)
    xrec_scratch[pl.ds(i * M, M), :] = xr


def _knn_step(j, xrec_scratch, loss_ref):
    xr = xrec_scratch[...]
    rows = xrec_scratch[pl.ds(j * ROW_BLK, ROW_BLK), :]
    sq_all = jnp.sum(xr * xr, axis=1)[None, :]
    sq_rows = jnp.sum(rows * rows, axis=1)[:, None]
    prod = jax.lax.dot_general(rows, xr, (((1,), (1,)), ((), ())),
                               preferred_element_type=F32)
    d = sq_rows + sq_all - 2.0 * prod
    total = jnp.zeros((), F32)
    for t in range(KNN):
        m = jnp.min(d, axis=1, keepdims=True)
        total = total + jnp.sum(m)
        if t + 1 < KNN:
            d = jnp.where(d == m, jnp.inf, d)

    @pl.when(j == 0)
    def _():
        loss_ref[...] = jnp.zeros((1, 1), F32)

    loss_ref[...] += jnp.reshape(total, (1, 1))


def kernel(x, W_in, b_in, Wq, bq, Wk, bk, Wv, bv, Wo, bo, W1, b1, W2, b2,
           g1, be1, g2, be2, W_down, b_down):
    scale = 1.0 / math.sqrt(DH)
    Wqkv = jnp.concatenate([Wq * scale, Wk, Wv], axis=2)
    bqkv = jnp.concatenate([bq * scale, bk, bv],
                           axis=1)[:, None, :]

    const2 = lambda b: (0, 0)
    const3 = lambda b: (0, 0, 0)
    full2 = lambda a: pl.BlockSpec(a.shape, const2)
    clamp = lambda b: jnp.minimum(b, N_FWD_STEPS - 1)

    in_specs = [
        pl.BlockSpec((NB, S, D_IN), lambda b: (clamp(b), 0, 0)),
        full2(W_in),
        pl.BlockSpec((1, D_MODEL), const2),
        pl.BlockSpec((L, D_MODEL, 3 * D_MODEL), const3),
        pl.BlockSpec((L, 1, 3 * D_MODEL), const3),
        pl.BlockSpec((L, D_MODEL, D_MODEL), const3),
        pl.BlockSpec((L, 1, D_MODEL), const3),
        pl.BlockSpec((L, D_MODEL, D_FF), const3),
        pl.BlockSpec((L, 1, D_FF), const3),
        pl.BlockSpec((L, D_FF, D_MODEL), const3),
        pl.BlockSpec((L, 1, D_MODEL), const3),
        pl.BlockSpec((L, 1, D_MODEL), const3),
        pl.BlockSpec((L, 1, D_MODEL), const3),
        pl.BlockSpec((L, 1, D_MODEL), const3),
        pl.BlockSpec((L, 1, D_MODEL), const3),
        full2(W_down),
        pl.BlockSpec((1, D_IN), const2),
    ]

    x_rec_b, loss = pl.pallas_call(
        _fused_kernel,
        grid=(N_FWD_STEPS + NUM_ROW_BLKS,),
        in_specs=in_specs,
        out_specs=[
            pl.BlockSpec((NB, S, D_IN), lambda b: (clamp(b), 0, 0)),
            pl.BlockSpec((1, 1), lambda b: (0, 0)),
        ],
        out_shape=[
            jax.ShapeDtypeStruct((B, S, D_IN), F32),
            jax.ShapeDtypeStruct((1, 1), F32),
        ],
        scratch_shapes=[pltpu.VMEM((N, D_IN), F32)],
    )(x, W_in.astype(BF16), b_in[None, :], Wqkv.astype(BF16), bqkv,
      Wo.astype(BF16), bo[:, None, :], W1.astype(BF16), b1[:, None, :],
      W2.astype(BF16), b2[:, None, :], g1[:, None, :],
      be1[:, None, :], g2[:, None, :], be2[:, None, :],
      W_down.astype(BF16), b_down[None, :])

    return loss[0, 0], x_rec_b.reshape(N, D_IN)


# speedup vs baseline: 1.0642x; 1.0642x over previous
import math

import jax
import jax.numpy as jnp
from jax.experimental import pallas as pl
from jax.experimental.pallas import tpu as pltpu

B, S, D_IN, D_MODEL, H, L, KNN = 16, 128, 64, 256, 8, 2, 16
D_FF = 1024
DH = D_MODEL // H
N = B * S
ROW_BLK = 128
NUM_ROW_BLKS = N // ROW_BLK
F32 = jnp.float32
BF16 = jnp.bfloat16


def _bdot(a, b_ref_val):
    return jnp.dot(a, b_ref_val, preferred_element_type=F32)


def _layernorm(x, g, b):
    m = jnp.mean(x, axis=-1, keepdims=True)
    v = jnp.mean((x - m) ** 2, axis=-1, keepdims=True)
    return (x - m) * jax.lax.rsqrt(v + 1e-5) * g + b


NB = 4


N_FWD_STEPS = B // NB


def _fused_kernel(x_ref, win_ref, bin_ref, wqkv_ref, bqkv_ref, wo_ref, bo_ref,
                  w1_ref, b1_ref, w2_ref, b2_ref, g1_ref, be1_ref, g2_ref,
                  be2_ref, wdown_ref, bdown_ref, out_ref, loss_ref,
                  xrec_scratch):
    i = pl.program_id(0)

    @pl.when(i < N_FWD_STEPS)
    def _():
        _fwd_step(i, x_ref, win_ref, bin_ref, wqkv_ref, bqkv_ref, wo_ref,
                  bo_ref, w1_ref, b1_ref, w2_ref, b2_ref, g1_ref, be1_ref,
                  g2_ref, be2_ref, wdown_ref, bdown_ref, out_ref,
                  xrec_scratch)

    @pl.when(i >= N_FWD_STEPS)
    def _():
        _knn_step(i - N_FWD_STEPS, xrec_scratch, loss_ref)


def _fwd_step(i, x_ref, win_ref, bin_ref, wqkv_ref, bqkv_ref, wo_ref, bo_ref,
              w1_ref, b1_ref, w2_ref, b2_ref, g1_ref, be1_ref, g2_ref,
              be2_ref, wdown_ref, bdown_ref, out_ref, xrec_scratch):
    M = NB * S
    xb = x_ref[...].reshape(M, D_IN)
    h = _bdot(xb, win_ref[...]) + bin_ref[...]
    sel = (jax.lax.broadcasted_iota(jnp.int32, (H * S, H), 0) // S
           == jax.lax.broadcasted_iota(jnp.int32, (H * S, H), 1)).astype(F32)
    rep = (jax.lax.broadcasted_iota(jnp.int32, (H, D_MODEL), 0)
           == jax.lax.broadcasted_iota(jnp.int32, (H, D_MODEL), 1) // DH
           ).astype(F32)
    for l in range(L):
        qkv = _bdot(h, wqkv_ref[l]) + bqkv_ref[l]
        o_rows = []
        for b in range(NB):
            rs = slice(b * S, (b + 1) * S)
            es, avs = [], []
            for hh in range(H):
                qs = slice(hh * DH, (hh + 1) * DH)
                ks = slice(D_MODEL + hh * DH, D_MODEL + (hh + 1) * DH)
                vs = slice(2 * D_MODEL + hh * DH, 2 * D_MODEL + (hh + 1) * DH)
                qh, kh, vh = qkv[rs, qs], qkv[rs, ks], qkv[rs, vs]
                s = jax.lax.dot_general(qh, kh, (((1,), (1,)), ((), ())),
                                        preferred_element_type=F32)
                e = jnp.exp(s)
                es.append(e)
                avs.append(jnp.dot(e, vh, preferred_element_type=F32))
            e_cat = jnp.concatenate(es, axis=1)
            sums = jnp.dot(e_cat, sel, preferred_element_type=F32)
            r_rep = jnp.dot(1.0 / sums, rep,
                            preferred_element_type=F32)
            o_rows.append(jnp.concatenate(avs, axis=1) * r_rep)
        o = jnp.concatenate(o_rows, axis=0)
        h = _layernorm(h + _bdot(o, wo_ref[l]) + bo_ref[l],
                       g1_ref[l], be1_ref[l])
        ff = jnp.maximum(_bdot(h, w1_ref[l]) + b1_ref[l], 0.0)
        ff = _bdot(ff, w2_ref[l]) + b2_ref[l]
        h = _layernorm(h + ff, g2_ref[l], be2_ref[l])
    xr = _bdot(h, wdown_ref[...]) + bdown_ref[...]
    out_ref[...] = xr.reshape(NB, S, D_IN)
    xrec_scratch[pl.ds(i * M, M), :] = xr


def _knn_step(j, xrec_scratch, loss_ref):
    xr = xrec_scratch[...]
    rows = xrec_scratch[pl.ds(j * ROW_BLK, ROW_BLK), :]
    sq_all = jnp.sum(xr * xr, axis=1)[None, :]
    sq_rows = jnp.sum(rows * rows, axis=1)[:, None]
    prod = jax.lax.dot_general(rows, xr, (((1,), (1,)), ((), ())),
                               preferred_element_type=F32)
    d = sq_rows + sq_all - 2.0 * prod
    total = jnp.zeros((), F32)
    for t in range(KNN):
        m = jnp.min(d, axis=1, keepdims=True)
        total = total + jnp.sum(m)
        if t + 1 < KNN:
            d = jnp.where(d == m, jnp.inf, d)

    @pl.when(j == 0)
    def _():
        loss_ref[...] = jnp.zeros((1, 1), F32)

    loss_ref[...] += jnp.reshape(total, (1, 1))


def kernel(x, W_in, b_in, Wq, bq, Wk, bk, Wv, bv, Wo, bo, W1, b1, W2, b2,
           g1, be1, g2, be2, W_down, b_down):
    scale = 1.0 / math.sqrt(DH)
    Wqkv = jnp.concatenate([Wq * scale, Wk, Wv], axis=2)
    bqkv = jnp.concatenate([bq * scale, bk, bv],
                           axis=1)[:, None, :]

    const2 = lambda b: (0, 0)
    const3 = lambda b: (0, 0, 0)
    full2 = lambda a: pl.BlockSpec(a.shape, const2)
    clamp = lambda b: jnp.minimum(b, N_FWD_STEPS - 1)

    in_specs = [
        pl.BlockSpec((NB, S, D_IN), lambda b: (clamp(b), 0, 0)),
        full2(W_in),
        pl.BlockSpec((1, D_MODEL), const2),
        pl.BlockSpec((L, D_MODEL, 3 * D_MODEL), const3),
        pl.BlockSpec((L, 1, 3 * D_MODEL), const3),
        pl.BlockSpec((L, D_MODEL, D_MODEL), const3),
        pl.BlockSpec((L, 1, D_MODEL), const3),
        pl.BlockSpec((L, D_MODEL, D_FF), const3),
        pl.BlockSpec((L, 1, D_FF), const3),
        pl.BlockSpec((L, D_FF, D_MODEL), const3),
        pl.BlockSpec((L, 1, D_MODEL), const3),
        pl.BlockSpec((L, 1, D_MODEL), const3),
        pl.BlockSpec((L, 1, D_MODEL), const3),
        pl.BlockSpec((L, 1, D_MODEL), const3),
        pl.BlockSpec((L, 1, D_MODEL), const3),
        full2(W_down),
        pl.BlockSpec((1, D_IN), const2),
    ]

    x_rec_b, loss = pl.pallas_call(
        _fused_kernel,
        grid=(N_FWD_STEPS + NUM_ROW_BLKS,),
        in_specs=in_specs,
        out_specs=[
            pl.BlockSpec((NB, S, D_IN), lambda b: (clamp(b), 0, 0)),
            pl.BlockSpec((1, 1), lambda b: (0, 0)),
        ],
        out_shape=[
            jax.ShapeDtypeStruct((B, S, D_IN), F32),
            jax.ShapeDtypeStruct((1, 1), F32),
        ],
        scratch_shapes=[pltpu.VMEM((N, D_IN), F32)],
    )(x, W_in, b_in[None, :], Wqkv, bqkv, Wo, bo[:, None, :],
      W1, b1[:, None, :], W2, b2[:, None, :], g1[:, None, :],
      be1[:, None, :], g2[:, None, :], be2[:, None, :], W_down,
      b_down[None, :])

    return loss[0, 0], x_rec_b.reshape(N, D_IN)


# speedup vs baseline: 1.0720x; 1.0074x over previous
import math

import jax
import jax.numpy as jnp
from jax.experimental import pallas as pl
from jax.experimental.pallas import tpu as pltpu

B, S, D_IN, D_MODEL, H, L, KNN = 16, 128, 64, 256, 8, 2, 16
D_FF = 1024
DH = D_MODEL // H
N = B * S
ROW_BLK = 128
NUM_ROW_BLKS = N // ROW_BLK
F32 = jnp.float32
BF16 = jnp.bfloat16


def _bdot(a, b_ref_val):
    return jnp.dot(a, b_ref_val, preferred_element_type=F32)


def _layernorm(x, g, b):
    m = jnp.mean(x, axis=-1, keepdims=True)
    v = jnp.mean((x - m) ** 2, axis=-1, keepdims=True)
    return (x - m) * jax.lax.rsqrt(v + 1e-5) * g + b


NB = 4


N_FWD_STEPS = B // NB


def _fused_kernel(x_ref, win_ref, bin_ref, wq_ref, bq_ref, wk_ref, bk_ref,
                  wv_ref, bv_ref, wo_ref, bo_ref,
                  w1_ref, b1_ref, w2_ref, b2_ref, g1_ref, be1_ref, g2_ref,
                  be2_ref, wdown_ref, bdown_ref, out_ref, loss_ref,
                  xrec_scratch):
    i = pl.program_id(0)

    @pl.when(i < N_FWD_STEPS)
    def _():
        _fwd_step(i, x_ref, win_ref, bin_ref, wq_ref, bq_ref, wk_ref, bk_ref,
                  wv_ref, bv_ref, wo_ref,
                  bo_ref, w1_ref, b1_ref, w2_ref, b2_ref, g1_ref, be1_ref,
                  g2_ref, be2_ref, wdown_ref, bdown_ref, out_ref,
                  xrec_scratch)

    @pl.when(i >= N_FWD_STEPS)
    def _():
        _knn_step(i - N_FWD_STEPS, xrec_scratch, loss_ref)


def _fwd_step(i, x_ref, win_ref, bin_ref, wq_ref, bq_ref, wk_ref, bk_ref,
              wv_ref, bv_ref, wo_ref, bo_ref,
              w1_ref, b1_ref, w2_ref, b2_ref, g1_ref, be1_ref, g2_ref,
              be2_ref, wdown_ref, bdown_ref, out_ref, xrec_scratch):
    M = NB * S
    scale = 1.0 / math.sqrt(DH)
    xb = x_ref[...].reshape(M, D_IN)
    h = _bdot(xb, win_ref[...]) + bin_ref[...]
    sel = (jax.lax.broadcasted_iota(jnp.int32, (H * S, H), 0) // S
           == jax.lax.broadcasted_iota(jnp.int32, (H * S, H), 1)).astype(F32)
    rep = (jax.lax.broadcasted_iota(jnp.int32, (H, D_MODEL), 0)
           == jax.lax.broadcasted_iota(jnp.int32, (H, D_MODEL), 1) // DH
           ).astype(F32)
    for l in range(L):
        q = (_bdot(h, wq_ref[l]) + bq_ref[l]) * scale
        k = _bdot(h, wk_ref[l]) + bk_ref[l]
        v = _bdot(h, wv_ref[l]) + bv_ref[l]
        o_rows = []
        for b in range(NB):
            rs = slice(b * S, (b + 1) * S)
            es, avs = [], []
            for hh in range(H):
                cs = slice(hh * DH, (hh + 1) * DH)
                qh, kh, vh = q[rs, cs], k[rs, cs], v[rs, cs]
                s = jax.lax.dot_general(qh, kh, (((1,), (1,)), ((), ())),
                                        preferred_element_type=F32)
                e = jnp.exp(s)
                es.append(e)
                avs.append(jnp.dot(e, vh, preferred_element_type=F32))
            e_cat = jnp.concatenate(es, axis=1)
            sums = jnp.dot(e_cat, sel, preferred_element_type=F32)
            r_rep = jnp.dot(1.0 / sums, rep,
                            preferred_element_type=F32)
            o_rows.append(jnp.concatenate(avs, axis=1) * r_rep)
        o = jnp.concatenate(o_rows, axis=0)
        h = _layernorm(h + _bdot(o, wo_ref[l]) + bo_ref[l],
                       g1_ref[l], be1_ref[l])
        ff = jnp.maximum(_bdot(h, w1_ref[l]) + b1_ref[l], 0.0)
        ff = _bdot(ff, w2_ref[l]) + b2_ref[l]
        h = _layernorm(h + ff, g2_ref[l], be2_ref[l])
    xr = _bdot(h, wdown_ref[...]) + bdown_ref[...]
    out_ref[...] = xr.reshape(NB, S, D_IN)
    xrec_scratch[pl.ds(i * M, M), :] = xr


def _knn_step(j, xrec_scratch, loss_ref):
    xr = xrec_scratch[...]
    rows = xrec_scratch[pl.ds(j * ROW_BLK, ROW_BLK), :]
    sq_all = jnp.sum(xr * xr, axis=1)[None, :]
    sq_rows = jnp.sum(rows * rows, axis=1)[:, None]
    prod = jax.lax.dot_general(rows, xr, (((1,), (1,)), ((), ())),
                               preferred_element_type=F32)
    d = sq_rows + sq_all - 2.0 * prod
    lo = jnp.minimum(d[:, :N // 2], d[:, N // 2:])
    hi = jnp.maximum(d[:, :N // 2], d[:, N // 2:])
    total = jnp.zeros((), F32)
    for t in range(KNN):
        m = jnp.min(lo, axis=1, keepdims=True)
        total = total + jnp.sum(m)
        if t + 1 < KNN:
            cond = lo == m
            lo = jnp.where(cond, hi, lo)
            hi = jnp.where(cond, jnp.inf, hi)

    @pl.when(j == 0)
    def _():
        loss_ref[...] = jnp.zeros((1, 1), F32)

    loss_ref[...] += jnp.reshape(total, (1, 1))


def kernel(x, W_in, b_in, Wq, bq, Wk, bk, Wv, bv, Wo, bo, W1, b1, W2, b2,
           g1, be1, g2, be2, W_down, b_down):
    const2 = lambda b: (0, 0)
    const3 = lambda b: (0, 0, 0)
    full2 = lambda a: pl.BlockSpec(a.shape, const2)
    clamp = lambda b: jnp.minimum(b, N_FWD_STEPS - 1)

    in_specs = [
        pl.BlockSpec((NB, S, D_IN), lambda b: (clamp(b), 0, 0)),
        full2(W_in),
        pl.BlockSpec((1, D_MODEL), const2),
        pl.BlockSpec((L, D_MODEL, D_MODEL), const3),
        pl.BlockSpec((L, 1, D_MODEL), const3),
        pl.BlockSpec((L, D_MODEL, D_MODEL), const3),
        pl.BlockSpec((L, 1, D_MODEL), const3),
        pl.BlockSpec((L, D_MODEL, D_MODEL), const3),
        pl.BlockSpec((L, 1, D_MODEL), const3),
        pl.BlockSpec((L, D_MODEL, D_MODEL), const3),
        pl.BlockSpec((L, 1, D_MODEL), const3),
        pl.BlockSpec((L, D_MODEL, D_FF), const3),
        pl.BlockSpec((L, 1, D_FF), const3),
        pl.BlockSpec((L, D_FF, D_MODEL), const3),
        pl.BlockSpec((L, 1, D_MODEL), const3),
        pl.BlockSpec((L, 1, D_MODEL), const3),
        pl.BlockSpec((L, 1, D_MODEL), const3),
        pl.BlockSpec((L, 1, D_MODEL), const3),
        pl.BlockSpec((L, 1, D_MODEL), const3),
        full2(W_down),
        pl.BlockSpec((1, D_IN), const2),
    ]

    x_rec_b, loss = pl.pallas_call(
        _fused_kernel,
        grid=(N_FWD_STEPS + NUM_ROW_BLKS,),
        in_specs=in_specs,
        out_specs=[
            pl.BlockSpec((NB, S, D_IN), lambda b: (clamp(b), 0, 0)),
            pl.BlockSpec((1, 1), lambda b: (0, 0)),
        ],
        out_shape=[
            jax.ShapeDtypeStruct((B, S, D_IN), F32),
            jax.ShapeDtypeStruct((1, 1), F32),
        ],
        scratch_shapes=[pltpu.VMEM((N, D_IN), F32)],
    )(x, W_in, b_in[None, :], Wq, bq[:, None, :], Wk, bk[:, None, :],
      Wv, bv[:, None, :], Wo, bo[:, None, :],
      W1, b1[:, None, :], W2, b2[:, None, :], g1[:, None, :],
      be1[:, None, :], g2[:, None, :], be2[:, None, :], W_down,
      b_down[None, :])

    return loss[0, 0], x_rec_b.reshape(N, D_IN)


# speedup vs baseline: 1.1103x; 1.0357x over previous
import math

import jax
import jax.numpy as jnp
from jax.experimental import pallas as pl
from jax.experimental.pallas import tpu as pltpu

B, S, D_IN, D_MODEL, H, L, KNN = 16, 128, 64, 256, 8, 2, 16
D_FF = 1024
DH = D_MODEL // H
N = B * S
ROW_BLK = 128
NUM_ROW_BLKS = N // ROW_BLK
F32 = jnp.float32
BF16 = jnp.bfloat16


def _bdot(a, b_ref_val):
    return jnp.dot(a, b_ref_val, preferred_element_type=F32)


def _layernorm(x, g, b):
    m = jnp.mean(x, axis=-1, keepdims=True)
    v = jnp.mean((x - m) ** 2, axis=-1, keepdims=True)
    return (x - m) * jax.lax.rsqrt(v + 1e-5) * g + b


NB = 8


N_FWD_STEPS = B // NB


def _fused_kernel(x_ref, win_ref, bin_ref, wq_ref, bq_ref, wk_ref, bk_ref,
                  wv_ref, bv_ref, wo_ref, bo_ref,
                  w1_ref, b1_ref, w2_ref, b2_ref, g1_ref, be1_ref, g2_ref,
                  be2_ref, wdown_ref, bdown_ref, out_ref, loss_ref,
                  xrec_scratch):
    i = pl.program_id(0)

    @pl.when(i < N_FWD_STEPS)
    def _():
        _fwd_step(i, x_ref, win_ref, bin_ref, wq_ref, bq_ref, wk_ref, bk_ref,
                  wv_ref, bv_ref, wo_ref,
                  bo_ref, w1_ref, b1_ref, w2_ref, b2_ref, g1_ref, be1_ref,
                  g2_ref, be2_ref, wdown_ref, bdown_ref, out_ref,
                  xrec_scratch)

    @pl.when(i >= N_FWD_STEPS)
    def _():
        _knn_step(i - N_FWD_STEPS, xrec_scratch, loss_ref)


def _fwd_step(i, x_ref, win_ref, bin_ref, wq_ref, bq_ref, wk_ref, bk_ref,
              wv_ref, bv_ref, wo_ref, bo_ref,
              w1_ref, b1_ref, w2_ref, b2_ref, g1_ref, be1_ref, g2_ref,
              be2_ref, wdown_ref, bdown_ref, out_ref, xrec_scratch):
    M = NB * S
    scale = 1.0 / math.sqrt(DH)
    xb = x_ref[...].reshape(M, D_IN)
    h = _bdot(xb, win_ref[...]) + bin_ref[...]
    sel = (jax.lax.broadcasted_iota(jnp.int32, (H * S, H), 0) // S
           == jax.lax.broadcasted_iota(jnp.int32, (H * S, H), 1)).astype(F32)
    rep = (jax.lax.broadcasted_iota(jnp.int32, (H, D_MODEL), 0)
           == jax.lax.broadcasted_iota(jnp.int32, (H, D_MODEL), 1) // DH
           ).astype(F32)
    for l in range(L):
        q = (_bdot(h, wq_ref[l]) + bq_ref[l]) * scale
        k = _bdot(h, wk_ref[l]) + bk_ref[l]
        v = _bdot(h, wv_ref[l]) + bv_ref[l]
        o_rows = []
        for b in range(NB):
            rs = slice(b * S, (b + 1) * S)
            es, avs = [], []
            for hh in range(H):
                cs = slice(hh * DH, (hh + 1) * DH)
                qh, kh, vh = q[rs, cs], k[rs, cs], v[rs, cs]
                s = jax.lax.dot_general(qh, kh, (((1,), (1,)), ((), ())),
                                        preferred_element_type=F32)
                e = jnp.exp(s)
                es.append(e)
                avs.append(jnp.dot(e, vh, preferred_element_type=F32))
            e_cat = jnp.concatenate(es, axis=1)
            sums = jnp.dot(e_cat, sel, preferred_element_type=F32)
            r_rep = jnp.dot(1.0 / sums, rep,
                            preferred_element_type=F32)
            o_rows.append(jnp.concatenate(avs, axis=1) * r_rep)
        o = jnp.concatenate(o_rows, axis=0)
        h = _layernorm(h + _bdot(o, wo_ref[l]) + bo_ref[l],
                       g1_ref[l], be1_ref[l])
        ff = jnp.maximum(_bdot(h, w1_ref[l]) + b1_ref[l], 0.0)
        ff = _bdot(ff, w2_ref[l]) + b2_ref[l]
        h = _layernorm(h + ff, g2_ref[l], be2_ref[l])
    xr = _bdot(h, wdown_ref[...]) + bdown_ref[...]
    out_ref[...] = xr.reshape(NB, S, D_IN)
    xrec_scratch[pl.ds(i * M, M), :] = xr


def _knn_step(j, xrec_scratch, loss_ref):
    xr = xrec_scratch[...]
    rows = xrec_scratch[pl.ds(j * ROW_BLK, ROW_BLK), :]
    sq_all = jnp.sum(xr * xr, axis=1)[None, :]
    sq_rows = jnp.sum(rows * rows, axis=1)[:, None]
    prod = jax.lax.dot_general(rows, xr, (((1,), (1,)), ((), ())),
                               preferred_element_type=F32)
    d = sq_rows + sq_all - 2.0 * prod
    lo = jnp.minimum(d[:, :N // 2], d[:, N // 2:])
    hi = jnp.maximum(d[:, :N // 2], d[:, N // 2:])
    total = jnp.zeros((), F32)
    for t in range(KNN):
        m = jnp.min(lo, axis=1, keepdims=True)
        total = total + jnp.sum(m)
        if t + 1 < KNN:
            cond = lo == m
            lo = jnp.where(cond, hi, lo)
            hi = jnp.where(cond, jnp.inf, hi)

    @pl.when(j == 0)
    def _():
        loss_ref[...] = jnp.zeros((1, 1), F32)

    loss_ref[...] += jnp.reshape(total, (1, 1))


def kernel(x, W_in, b_in, Wq, bq, Wk, bk, Wv, bv, Wo, bo, W1, b1, W2, b2,
           g1, be1, g2, be2, W_down, b_down):
    const2 = lambda b: (0, 0)
    const3 = lambda b: (0, 0, 0)
    full2 = lambda a: pl.BlockSpec(a.shape, const2)
    clamp = lambda b: jnp.minimum(b, N_FWD_STEPS - 1)

    in_specs = [
        pl.BlockSpec((NB, S, D_IN), lambda b: (clamp(b), 0, 0)),
        full2(W_in),
        pl.BlockSpec((1, D_MODEL), const2),
        pl.BlockSpec((L, D_MODEL, D_MODEL), const3),
        pl.BlockSpec((L, 1, D_MODEL), const3),
        pl.BlockSpec((L, D_MODEL, D_MODEL), const3),
        pl.BlockSpec((L, 1, D_MODEL), const3),
        pl.BlockSpec((L, D_MODEL, D_MODEL), const3),
        pl.BlockSpec((L, 1, D_MODEL), const3),
        pl.BlockSpec((L, D_MODEL, D_MODEL), const3),
        pl.BlockSpec((L, 1, D_MODEL), const3),
        pl.BlockSpec((L, D_MODEL, D_FF), const3),
        pl.BlockSpec((L, 1, D_FF), const3),
        pl.BlockSpec((L, D_FF, D_MODEL), const3),
        pl.BlockSpec((L, 1, D_MODEL), const3),
        pl.BlockSpec((L, 1, D_MODEL), const3),
        pl.BlockSpec((L, 1, D_MODEL), const3),
        pl.BlockSpec((L, 1, D_MODEL), const3),
        pl.BlockSpec((L, 1, D_MODEL), const3),
        full2(W_down),
        pl.BlockSpec((1, D_IN), const2),
    ]

    x_rec_b, loss = pl.pallas_call(
        _fused_kernel,
        grid=(N_FWD_STEPS + NUM_ROW_BLKS,),
        in_specs=in_specs,
        out_specs=[
            pl.BlockSpec((NB, S, D_IN), lambda b: (clamp(b), 0, 0)),
            pl.BlockSpec((1, 1), lambda b: (0, 0)),
        ],
        out_shape=[
            jax.ShapeDtypeStruct((B, S, D_IN), F32),
            jax.ShapeDtypeStruct((1, 1), F32),
        ],
        scratch_shapes=[pltpu.VMEM((N, D_IN), F32)],
    )(x, W_in, b_in[None, :], Wq, bq[:, None, :], Wk, bk[:, None, :],
      Wv, bv[:, None, :], Wo, bo[:, None, :],
      W1, b1[:, None, :], W2, b2[:, None, :], g1[:, None, :],
      be1[:, None, :], g2[:, None, :], be2[:, None, :], W_down,
      b_down[None, :])

    return loss[0, 0], x_rec_b.reshape(N, D_IN)


# speedup vs baseline: 1.1272x; 1.0153x over previous
import math

import jax
import jax.numpy as jnp
from jax.experimental import pallas as pl
from jax.experimental.pallas import tpu as pltpu

B, S, D_IN, D_MODEL, H, L, KNN = 16, 128, 64, 256, 8, 2, 16
D_FF = 1024
DH = D_MODEL // H
N = B * S
ROW_BLK = 128
NUM_ROW_BLKS = N // ROW_BLK
F32 = jnp.float32
BF16 = jnp.bfloat16


def _bdot(a, b_ref_val):
    return jnp.dot(a, b_ref_val, preferred_element_type=F32)


def _layernorm(x, g, b):
    m = jnp.mean(x, axis=-1, keepdims=True)
    v = jnp.mean((x - m) ** 2, axis=-1, keepdims=True)
    return (x - m) * jax.lax.rsqrt(v + 1e-5) * g + b


NB = 8


N_FWD_STEPS = B // NB


def _fused_kernel(x_ref, win_ref, bin_ref, wq_ref, bq_ref, wk_ref, bk_ref,
                  wv_ref, bv_ref, wo_ref, bo_ref,
                  w1_ref, b1_ref, w2_ref, b2_ref, g1_ref, be1_ref, g2_ref,
                  be2_ref, wdown_ref, bdown_ref, out_ref, loss_ref,
                  xrec_scratch, sq_scratch):
    i = pl.program_id(0)

    @pl.when(i < N_FWD_STEPS)
    def _():
        _fwd_step(i, x_ref, win_ref, bin_ref, wq_ref, bq_ref, wk_ref, bk_ref,
                  wv_ref, bv_ref, wo_ref,
                  bo_ref, w1_ref, b1_ref, w2_ref, b2_ref, g1_ref, be1_ref,
                  g2_ref, be2_ref, wdown_ref, bdown_ref, out_ref,
                  xrec_scratch)

    @pl.when(i >= N_FWD_STEPS)
    def _():
        _knn_step(i - N_FWD_STEPS, xrec_scratch, sq_scratch, loss_ref)


def _fwd_step(i, x_ref, win_ref, bin_ref, wq_ref, bq_ref, wk_ref, bk_ref,
              wv_ref, bv_ref, wo_ref, bo_ref,
              w1_ref, b1_ref, w2_ref, b2_ref, g1_ref, be1_ref, g2_ref,
              be2_ref, wdown_ref, bdown_ref, out_ref, xrec_scratch):
    M = NB * S
    scale = 1.0 / math.sqrt(DH)
    xb = x_ref[...].reshape(M, D_IN)
    h = _bdot(xb, win_ref[...]) + bin_ref[...]
    sel = (jax.lax.broadcasted_iota(jnp.int32, (H * S, H), 0) // S
           == jax.lax.broadcasted_iota(jnp.int32, (H * S, H), 1)).astype(F32)
    rep = (jax.lax.broadcasted_iota(jnp.int32, (H, D_MODEL), 0)
           == jax.lax.broadcasted_iota(jnp.int32, (H, D_MODEL), 1) // DH
           ).astype(F32)
    for l in range(L):
        q = (_bdot(h, wq_ref[l]) + bq_ref[l]) * scale
        k = _bdot(h, wk_ref[l]) + bk_ref[l]
        v = _bdot(h, wv_ref[l]) + bv_ref[l]
        o_rows = []
        for b in range(NB):
            rs = slice(b * S, (b + 1) * S)
            es, avs = [], []
            for hh in range(H):
                cs = slice(hh * DH, (hh + 1) * DH)
                qh, kh, vh = q[rs, cs], k[rs, cs], v[rs, cs]
                s = jax.lax.dot_general(qh, kh, (((1,), (1,)), ((), ())),
                                        preferred_element_type=F32)
                e = jnp.exp(s)
                es.append(e)
                avs.append(jnp.dot(e, vh, preferred_element_type=F32))
            e_cat = jnp.concatenate(es, axis=1)
            sums = jnp.dot(e_cat, sel, preferred_element_type=F32)
            r_rep = jnp.dot(1.0 / sums, rep,
                            preferred_element_type=F32)
            o_rows.append(jnp.concatenate(avs, axis=1) * r_rep)
        o = jnp.concatenate(o_rows, axis=0)
        h = _layernorm(h + _bdot(o, wo_ref[l]) + bo_ref[l],
                       g1_ref[l], be1_ref[l])
        ff = jnp.maximum(_bdot(h, w1_ref[l]) + b1_ref[l], 0.0)
        ff = _bdot(ff, w2_ref[l]) + b2_ref[l]
        h = _layernorm(h + ff, g2_ref[l], be2_ref[l])
    xr = _bdot(h, wdown_ref[...]) + bdown_ref[...]
    out_ref[...] = xr.reshape(NB, S, D_IN)
    xrec_scratch[pl.ds(i * M, M), :] = xr


def _knn_step(j, xrec_scratch, sq_scratch, loss_ref):
    xr = xrec_scratch[...]
    rows = xrec_scratch[pl.ds(j * ROW_BLK, ROW_BLK), :]

    @pl.when(j == 0)
    def _():
        sq_scratch[...] = jnp.sum(xr * xr, axis=1)[None, :]

    sq_all = sq_scratch[...]
    sq_rows = jnp.sum(rows * rows, axis=1)[:, None]
    prod = jax.lax.dot_general(rows, xr, (((1,), (1,)), ((), ())),
                               preferred_element_type=F32)
    d = sq_rows + sq_all - 2.0 * prod
    lo = jnp.minimum(d[:, :N // 2], d[:, N // 2:])
    hi = jnp.maximum(d[:, :N // 2], d[:, N // 2:])
    total = jnp.zeros((), F32)
    for t in range(KNN):
        m = jnp.min(lo, axis=1, keepdims=True)
        total = total + jnp.sum(m)
        if t + 1 < KNN:
            cond = lo == m
            lo = jnp.where(cond, hi, lo)
            hi = jnp.where(cond, jnp.inf, hi)

    @pl.when(j == 0)
    def _():
        loss_ref[...] = jnp.zeros((1, 1), F32)

    loss_ref[...] += jnp.reshape(total, (1, 1))


def kernel(x, W_in, b_in, Wq, bq, Wk, bk, Wv, bv, Wo, bo, W1, b1, W2, b2,
           g1, be1, g2, be2, W_down, b_down):
    const2 = lambda b: (0, 0)
    const3 = lambda b: (0, 0, 0)
    full2 = lambda a: pl.BlockSpec(a.shape, const2)
    clamp = lambda b: jnp.minimum(b, N_FWD_STEPS - 1)

    in_specs = [
        pl.BlockSpec((NB, S, D_IN), lambda b: (clamp(b), 0, 0)),
        full2(W_in),
        pl.BlockSpec((1, D_MODEL), const2),
        pl.BlockSpec((L, D_MODEL, D_MODEL), const3),
        pl.BlockSpec((L, 1, D_MODEL), const3),
        pl.BlockSpec((L, D_MODEL, D_MODEL), const3),
        pl.BlockSpec((L, 1, D_MODEL), const3),
        pl.BlockSpec((L, D_MODEL, D_MODEL), const3),
        pl.BlockSpec((L, 1, D_MODEL), const3),
        pl.BlockSpec((L, D_MODEL, D_MODEL), const3),
        pl.BlockSpec((L, 1, D_MODEL), const3),
        pl.BlockSpec((L, D_MODEL, D_FF), const3),
        pl.BlockSpec((L, 1, D_FF), const3),
        pl.BlockSpec((L, D_FF, D_MODEL), const3),
        pl.BlockSpec((L, 1, D_MODEL), const3),
        pl.BlockSpec((L, 1, D_MODEL), const3),
        pl.BlockSpec((L, 1, D_MODEL), const3),
        pl.BlockSpec((L, 1, D_MODEL), const3),
        pl.BlockSpec((L, 1, D_MODEL), const3),
        full2(W_down),
        pl.BlockSpec((1, D_IN), const2),
    ]

    x_rec_b, loss = pl.pallas_call(
        _fused_kernel,
        grid=(N_FWD_STEPS + NUM_ROW_BLKS,),
        in_specs=in_specs,
        out_specs=[
            pl.BlockSpec((NB, S, D_IN), lambda b: (clamp(b), 0, 0)),
            pl.BlockSpec((1, 1), lambda b: (0, 0)),
        ],
        out_shape=[
            jax.ShapeDtypeStruct((B, S, D_IN), F32),
            jax.ShapeDtypeStruct((1, 1), F32),
        ],
        scratch_shapes=[pltpu.VMEM((N, D_IN), F32),
                        pltpu.VMEM((1, N), F32)],
    )(x, W_in, b_in[None, :], Wq, bq[:, None, :], Wk, bk[:, None, :],
      Wv, bv[:, None, :], Wo, bo[:, None, :],
      W1, b1[:, None, :], W2, b2[:, None, :], g1[:, None, :],
      be1[:, None, :], g2[:, None, :], be2[:, None, :], W_down,
      b_down[None, :])

    return loss[0, 0], x_rec_b.reshape(N, D_IN)
